# baseline (speedup 1.0000x reference)
import jax
import jax.numpy as jnp
from jax.experimental import pallas as pl
from jax.experimental.pallas import tpu as pltpu

_TOPK = 2
_BM = 1024


def _router_kernel(x_ref, w_ref, idx_ref, gate_ref, lbuf):
    i = pl.program_id(0)
    n = pl.num_programs(0)

    @pl.when(i > 0)
    def _reduce_prev():
        logits = lbuf[...]
        e = logits.shape[0]
        row = jax.lax.broadcasted_iota(jnp.int32, logits.shape, 0)
        l1 = jnp.max(logits, axis=0, keepdims=True)
        i1 = jnp.min(jnp.where(logits == l1, row, e),
                     axis=0, keepdims=True)
        masked = jnp.where(row == i1, -jnp.inf, logits)
        l2 = jnp.max(masked, axis=0, keepdims=True)
        i2 = jnp.min(jnp.where(masked == l2, row, e),
                     axis=0, keepdims=True)
        e2 = jnp.exp(l2 - l1)
        denom = 1.0 + e2
        idx_ref[...] = jnp.concatenate([i1, i2], axis=0)
        gate_ref[...] = jnp.concatenate([1.0 / denom, e2 / denom], axis=0)

    @pl.when(i < n - 1)
    def _matmul_cur():
        lbuf[...] = jax.lax.dot_general(
            w_ref[...], x_ref[...], (((1,), (1,)), ((), ())),
            preferred_element_type=jnp.float32)


@jax.jit
def kernel(x, W):
    m, k = x.shape
    e = W.shape[0]
    nblocks = m // _BM
    idx_t, gates_t = pl.pallas_call(
        _router_kernel,
        grid=(nblocks + 1,),
        in_specs=[
            pl.BlockSpec((_BM, k), lambda i: (jnp.minimum(i, nblocks - 1), 0)),
            pl.BlockSpec((e, k), lambda i: (0, 0)),
        ],
        out_specs=[
            pl.BlockSpec((_TOPK, _BM), lambda i: (0, jnp.maximum(i - 1, 0))),
            pl.BlockSpec((_TOPK, _BM), lambda i: (0, jnp.maximum(i - 1, 0))),
        ],
        out_shape=[
            jax.ShapeDtypeStruct((_TOPK, m), jnp.int32),
            jax.ShapeDtypeStruct((_TOPK, m), jnp.float32),
        ],
        scratch_shapes=[
            pltpu.VMEM((e, _BM), jnp.float32),
        ],
    )(x, W)
    return idx_t.T, gates_t.T
